# baseline (speedup 1.0000x reference)
import functools

import jax
import jax.numpy as jnp
from jax import lax
from jax.experimental import pallas as pl
from jax.experimental.pallas import tpu as pltpu
from jax.experimental.pallas import tpu_sc as plsc

N_HEADS = 16
N_BUCKETS = 32
MAX_DIST = 128
Q_LEN = 2048
K_LEN = 2048

NSHIFT = 8
TBL = 4096
LAG = 8


def _table_body(emb_ref, log16_ref, doff_ref, out_ref):
    r = lax.broadcasted_iota(jnp.int32, (NSHIFT, TBL), 0)
    c = lax.broadcasted_iota(jnp.int32, (NSHIFT, TBL), 1)
    relative_position = ((NSHIFT - 1 - r) + c) - (Q_LEN - 1) + doff_ref[0, 0]
    n = -relative_position
    half = N_BUCKETS // 2
    big = jnp.where(n < 0, half, 0)
    n = jnp.abs(n)
    max_exact = half // 2
    nf = n.astype(jnp.float32)
    val_large = max_exact + (
        jnp.log(nf / max_exact) / log16_ref[0, 0] * (half - max_exact)
    ).astype(jnp.int32)
    val_large = jnp.minimum(val_large, half - 1)
    bucket = big + jnp.where(n < max_exact, n, val_large)
    for h in range(N_HEADS):
        acc = jnp.zeros((NSHIFT, TBL), jnp.float32)
        for b in range(N_BUCKETS):
            acc = acc + jnp.where(bucket == b, emb_ref[h, b], 0.0)
        out_ref[h] = acc


def _make_table(rel_embedding, log16, d_off):
    return pl.pallas_call(
        _table_body,
        out_shape=jax.ShapeDtypeStruct((N_HEADS, NSHIFT, TBL), jnp.float32),
        in_specs=[
            pl.BlockSpec(memory_space=pltpu.SMEM),
            pl.BlockSpec(memory_space=pltpu.SMEM),
            pl.BlockSpec(memory_space=pltpu.SMEM),
        ],
    )(rel_embedding, log16, d_off)


def _expand_body(tbl_hbm, out_hbm, sh_vmem, sem):
    h = lax.axis_index("s")
    half = lax.axis_index("c")
    rows = Q_LEN // 2
    base = half * rows

    pltpu.sync_copy(tbl_hbm.at[h], sh_vmem)

    def body(b, carry):
        row0 = base + b * NSHIFT
        row0 = pl.multiple_of(row0, NSHIFT)
        c0 = (Q_LEN - NSHIFT) - row0
        c0 = pl.multiple_of(c0, NSHIFT)
        pltpu.make_async_copy(
            sh_vmem.at[:, pl.ds(c0, K_LEN)],
            out_hbm.at[h, pl.ds(row0, NSHIFT), :],
            sem,
        ).start()

        @pl.when(b >= LAG)
        def _():
            pltpu.make_async_copy(
                sh_vmem.at[:, pl.ds(0, K_LEN)],
                out_hbm.at[h, pl.ds(base, NSHIFT), :],
                sem,
            ).wait()

        return carry

    lax.fori_loop(0, rows // NSHIFT, body, 0)
    for _ in range(LAG):
        pltpu.make_async_copy(
            sh_vmem.at[:, pl.ds(0, K_LEN)],
            out_hbm.at[h, pl.ds(base, NSHIFT), :],
            sem,
        ).wait()


@functools.cache
def _make_expand():
    return pl.kernel(
        _expand_body,
        out_type=jax.ShapeDtypeStruct((N_HEADS, Q_LEN, K_LEN), jnp.float32),
        mesh=plsc.VectorSubcoreMesh(core_axis_name="c", subcore_axis_name="s"),
        scratch_types=[
            pltpu.VMEM((NSHIFT, TBL), jnp.float32),
            pltpu.SemaphoreType.DMA,
        ],
        compiler_params=pltpu.CompilerParams(use_tc_tiling_on_sc=False),
    )


def kernel(query_length, key_length, rel_embedding):
    d_off = jnp.asarray(
        (key_length - K_LEN) - (query_length - Q_LEN), jnp.int32
    ).reshape(1, 1)
    log16 = jnp.log(jnp.full((1, 1), MAX_DIST / (N_BUCKETS // 4), jnp.float32))
    tbl = _make_table(rel_embedding, log16, d_off)
    return _make_expand()(tbl)


# speedup vs baseline: 1.8061x; 1.8061x over previous
import functools

import jax
import jax.numpy as jnp
from jax import lax
from jax.experimental import pallas as pl
from jax.experimental.pallas import tpu as pltpu
from jax.experimental.pallas import tpu_sc as plsc

N_HEADS = 16
N_BUCKETS = 32
MAX_DIST = 128
Q_LEN = 2048
K_LEN = 2048

NSHIFT = 8
NVER = 16
TBL = 4096
TBLV = TBL - MAX_DIST
LAG = 8


def _table_body(emb_ref, log16_ref, doff_ref, out_ref):
    h = pl.program_id(0)
    r = lax.broadcasted_iota(jnp.int32, (NSHIFT, TBL), 0)
    c = lax.broadcasted_iota(jnp.int32, (NSHIFT, TBL), 1)
    relative_position = ((NSHIFT - 1 - r) + c) - (Q_LEN - 1) + doff_ref[0, 0]
    n = -relative_position
    half = N_BUCKETS // 2
    big = jnp.where(n < 0, half, 0)
    n = jnp.abs(n)
    max_exact = half // 2
    nf = n.astype(jnp.float32)
    val_large = max_exact + (
        jnp.log(nf / max_exact) / log16_ref[0, 0] * (half - max_exact)
    ).astype(jnp.int32)
    val_large = jnp.minimum(val_large, half - 1)
    bucket = big + jnp.where(n < max_exact, n, val_large)
    acc = jnp.zeros((NSHIFT, TBL), jnp.float32)
    for b in range(N_BUCKETS):
        acc = acc + jnp.where(bucket == b, emb_ref[h, b], 0.0)
    for k in range(NVER):
        out_ref[0, k] = lax.slice(acc, (0, 8 * k), (NSHIFT, 8 * k + TBLV))


def _make_table(rel_embedding, log16, d_off):
    return pl.pallas_call(
        _table_body,
        out_shape=jax.ShapeDtypeStruct((N_HEADS, NVER, NSHIFT, TBLV), jnp.float32),
        grid=(N_HEADS,),
        in_specs=[
            pl.BlockSpec(memory_space=pltpu.SMEM),
            pl.BlockSpec(memory_space=pltpu.SMEM),
            pl.BlockSpec(memory_space=pltpu.SMEM),
        ],
        out_specs=pl.BlockSpec(
            (1, NVER, NSHIFT, TBLV), lambda h: (h, 0, 0, 0)
        ),
    )(rel_embedding, log16, d_off)


def _expand_body(tbl_hbm, out_hbm, spmem, sem):
    core = lax.axis_index("c")
    tile = lax.axis_index("s")
    nhpc = N_HEADS // 2

    def do_head(idx, carry):
        h = core * nhpc + idx

        @pl.when(tile == 0)
        def _():
            pltpu.sync_copy(tbl_hbm.at[h], spmem)
        plsc.subcore_barrier()

        def blk(m, c2):
            row0 = 128 * tile + NSHIFT * m
            row0 = pl.multiple_of(row0, NSHIFT)
            c0 = (Q_LEN - NSHIFT) - row0
            kp = (c0 >> 3) & (NVER - 1)
            col0 = (c0 >> 7) * 128
            col0 = pl.multiple_of(col0, 128)
            pltpu.make_async_copy(
                spmem.at[kp, :, pl.ds(col0, K_LEN)],
                out_hbm.at[h, pl.ds(row0, NSHIFT), :],
                sem,
            ).start()

            @pl.when(m >= LAG)
            def _():
                pltpu.make_async_copy(
                    spmem.at[0, :, pl.ds(0, K_LEN)],
                    out_hbm.at[h, pl.ds(0, NSHIFT), :],
                    sem,
                ).wait()

            return c2

        lax.fori_loop(0, 128 // NSHIFT, blk, 0)
        for _ in range(LAG):
            pltpu.make_async_copy(
                spmem.at[0, :, pl.ds(0, K_LEN)],
                out_hbm.at[h, pl.ds(0, NSHIFT), :],
                sem,
            ).wait()
        plsc.subcore_barrier()
        return carry

    lax.fori_loop(0, nhpc, do_head, 0)


@functools.cache
def _make_expand():
    return pl.kernel(
        _expand_body,
        out_type=jax.ShapeDtypeStruct((N_HEADS, Q_LEN, K_LEN), jnp.float32),
        mesh=plsc.VectorSubcoreMesh(core_axis_name="c", subcore_axis_name="s"),
        scratch_types=[
            pltpu.VMEM_SHARED((NVER, NSHIFT, TBLV), jnp.float32),
            pltpu.SemaphoreType.DMA,
        ],
    )


def kernel(query_length, key_length, rel_embedding):
    d_off = jnp.asarray(
        (key_length - K_LEN) - (query_length - Q_LEN), jnp.int32
    ).reshape(1, 1)
    log16 = jnp.log(jnp.full((1, 1), MAX_DIST / (N_BUCKETS // 4), jnp.float32))
    tbl = _make_table(rel_embedding, log16, d_off)
    return _make_expand()(tbl)


# speedup vs baseline: 2.0398x; 1.1294x over previous
import functools

import jax
import jax.numpy as jnp
from jax import lax
from jax.experimental import pallas as pl
from jax.experimental.pallas import tpu as pltpu
from jax.experimental.pallas import tpu_sc as plsc

N_HEADS = 16
N_BUCKETS = 32
MAX_DIST = 128
Q_LEN = 2048
K_LEN = 2048

NSHIFT = 8
NVER = 16
TBL = 4096
TBLV = TBL - MAX_DIST
LAG = 8


def _table_body(emb_ref, log16_ref, doff_ref, out_ref):
    h = pl.program_id(0)
    r = lax.broadcasted_iota(jnp.int32, (NSHIFT, TBL), 0)
    c = lax.broadcasted_iota(jnp.int32, (NSHIFT, TBL), 1)
    relative_position = ((NSHIFT - 1 - r) + c) - (Q_LEN - 1) + doff_ref[0, 0]
    n = -relative_position
    half = N_BUCKETS // 2
    big = jnp.where(n < 0, half, 0)
    n = jnp.abs(n)
    max_exact = half // 2
    nf = n.astype(jnp.float32)
    val_large = max_exact + (
        jnp.log(nf / max_exact) / log16_ref[0, 0] * (half - max_exact)
    ).astype(jnp.int32)
    val_large = jnp.minimum(val_large, half - 1)
    bucket = big + jnp.where(n < max_exact, n, val_large)
    acc = jnp.zeros((NSHIFT, TBL), jnp.float32)
    for b in range(N_BUCKETS):
        acc = acc + jnp.where(bucket == b, emb_ref[h, b], 0.0)
    for k in range(NVER):
        out_ref[0, k] = lax.slice(acc, (0, 8 * k), (NSHIFT, 8 * k + TBLV))


def _make_table(rel_embedding, log16, d_off):
    return pl.pallas_call(
        _table_body,
        out_shape=jax.ShapeDtypeStruct((N_HEADS, NVER, NSHIFT, TBLV), jnp.float32),
        grid=(N_HEADS,),
        in_specs=[
            pl.BlockSpec(memory_space=pltpu.SMEM),
            pl.BlockSpec(memory_space=pltpu.SMEM),
            pl.BlockSpec(memory_space=pltpu.SMEM),
        ],
        out_specs=pl.BlockSpec(
            (1, NVER, NSHIFT, TBLV), lambda h: (h, 0, 0, 0)
        ),
    )(rel_embedding, log16, d_off)


def _expand_body(tbl_hbm, out_hbm, spmem, sem, stage_sem):
    core = lax.axis_index("c")
    tile = lax.axis_index("s")
    nhpc = N_HEADS // 2

    @pl.when(tile == 0)
    def _():
        pltpu.sync_copy(tbl_hbm.at[core * nhpc], spmem.at[0])
    plsc.subcore_barrier()

    def do_head(idx, carry):
        h = core * nhpc + idx
        buf = idx & 1

        @pl.when((tile == 0) & (idx < nhpc - 1))
        def _():
            pltpu.make_async_copy(
                tbl_hbm.at[h + 1], spmem.at[1 - buf], stage_sem
            ).start()

        def blk(m, c2):
            row0 = 128 * tile + NSHIFT * m
            row0 = pl.multiple_of(row0, NSHIFT)
            c0 = (Q_LEN - NSHIFT) - row0
            kp = (c0 >> 3) & (NVER - 1)
            col0 = (c0 >> 7) * 128
            col0 = pl.multiple_of(col0, 128)
            pltpu.make_async_copy(
                spmem.at[buf, kp, :, pl.ds(col0, K_LEN)],
                out_hbm.at[h, pl.ds(row0, NSHIFT), :],
                sem,
            ).start()

            @pl.when(m >= LAG)
            def _():
                pltpu.make_async_copy(
                    spmem.at[0, 0, :, pl.ds(0, K_LEN)],
                    out_hbm.at[h, pl.ds(0, NSHIFT), :],
                    sem,
                ).wait()

            return c2

        lax.fori_loop(0, 128 // NSHIFT, blk, 0)
        for _ in range(LAG):
            pltpu.make_async_copy(
                spmem.at[0, 0, :, pl.ds(0, K_LEN)],
                out_hbm.at[h, pl.ds(0, NSHIFT), :],
                sem,
            ).wait()

        @pl.when((tile == 0) & (idx < nhpc - 1))
        def _():
            pltpu.make_async_copy(
                tbl_hbm.at[h + 1], spmem.at[1 - buf], stage_sem
            ).wait()

        plsc.subcore_barrier()
        return carry

    lax.fori_loop(0, nhpc, do_head, 0)


@functools.cache
def _make_expand():
    return pl.kernel(
        _expand_body,
        out_type=jax.ShapeDtypeStruct((N_HEADS, Q_LEN, K_LEN), jnp.float32),
        mesh=plsc.VectorSubcoreMesh(core_axis_name="c", subcore_axis_name="s"),
        scratch_types=[
            pltpu.VMEM_SHARED((2, NVER, NSHIFT, TBLV), jnp.float32),
            pltpu.SemaphoreType.DMA,
            pltpu.SemaphoreType.DMA,
        ],
    )


def kernel(query_length, key_length, rel_embedding):
    d_off = jnp.asarray(
        (key_length - K_LEN) - (query_length - Q_LEN), jnp.int32
    ).reshape(1, 1)
    log16 = jnp.log(jnp.full((1, 1), MAX_DIST / (N_BUCKETS // 4), jnp.float32))
    tbl = _make_table(rel_embedding, log16, d_off)
    return _make_expand()(tbl)
